# Initial kernel scaffold; baseline (speedup 1.0000x reference)
import jax
import jax.numpy as jnp
from jax.experimental import pallas as pl


def kernel(x):
    raise NotImplementedError("write your pallas kernel here")


# speedup vs baseline: 303.5107x; 303.5107x over previous
import functools

import jax
import jax.numpy as jnp
from jax import lax
from jax.experimental import pallas as pl
from jax.experimental.pallas import tpu as pltpu
from jax.experimental.pallas import tpu_sc as plsc

NB = 256
L = 16
NW = 32
CHUNK = 16384


def _quantize(v):
    t = jnp.minimum(jnp.maximum(v * float(NB - 1), 0.0), float(NB - 1))
    return t.astype(jnp.int32)


def _make_he(nimg, pix):
    ipw = nimg // NW
    nchunk = pix // CHUNK
    vpc = CHUNK // L
    scale = 1.0 / float(pix)
    mesh = plsc.VectorSubcoreMesh(core_axis_name="c", subcore_axis_name="s")

    @functools.partial(
        pl.kernel,
        mesh=mesh,
        out_type=jax.ShapeDtypeStruct((nimg * pix,), jnp.float32),
        compiler_params=pltpu.CompilerParams(needs_layout_passes=False),
        scratch_types=[
            pltpu.VMEM((CHUNK,), jnp.float32),
            pltpu.VMEM((CHUNK,), jnp.float32),
            pltpu.VMEM((L * NB,), jnp.float32),
            pltpu.VMEM((NB,), jnp.float32),
        ],
    )
    def he(x_hbm, out_hbm, inbuf, outbuf, hist, cdf):
        wid = lax.axis_index("s") * 2 + lax.axis_index("c")
        lane_base = lax.iota(jnp.int32, L) * NB
        ones = jnp.ones((L,), jnp.float32)
        zeros = jnp.zeros((L,), jnp.float32)

        for img in range(ipw):
            base = (wid * ipw + img) * pix

            def zrow(r, _):
                hist[pl.ds(r * L, L)] = zeros
                return 0
            lax.fori_loop(0, (L * NB) // L, zrow, 0)

            def chunk1(ch, _):
                pltpu.sync_copy(x_hbm.at[pl.ds(base + ch * CHUNK, CHUNK)],
                                inbuf)
                def vec(i, _):
                    q = _quantize(inbuf[pl.ds(i * L, L)])
                    plsc.addupdate_scatter(hist, [lane_base + q], ones)
                    return 0
                lax.fori_loop(0, vpc, vec, 0)
                return 0
            lax.fori_loop(0, nchunk, chunk1, 0)

            carry = jnp.float32(0.0)
            for g in range(NB // L):
                acc = hist[pl.ds(g * L, L)]
                for r in range(1, L):
                    acc = acc + hist[pl.ds(r * NB + g * L, L)]
                cs = jnp.cumsum(acc) + carry
                carry = jnp.max(cs)
                cdf[pl.ds(g * L, L)] = cs * scale

            def chunk2(ch, _):
                pltpu.sync_copy(x_hbm.at[pl.ds(base + ch * CHUNK, CHUNK)],
                                inbuf)
                def vec(i, _):
                    q = _quantize(inbuf[pl.ds(i * L, L)])
                    outbuf[pl.ds(i * L, L)] = plsc.load_gather(cdf, [q])
                    return 0
                lax.fori_loop(0, vpc, vec, 0)
                pltpu.sync_copy(outbuf,
                                out_hbm.at[pl.ds(base + ch * CHUNK, CHUNK)])
                return 0
            lax.fori_loop(0, nchunk, chunk2, 0)

    return he


def kernel(x):
    b, c, h, w = x.shape
    nimg, pix = b * c, h * w
    y = _make_he(nimg, pix)(x.reshape(nimg * pix))
    return y.reshape(b, c, h, w)


# speedup vs baseline: 957.8757x; 3.1560x over previous
import functools

import jax
import jax.numpy as jnp
from jax import lax
from jax.experimental import pallas as pl
from jax.experimental.pallas import tpu as pltpu
from jax.experimental.pallas import tpu_sc as plsc

NB = 256
L = 16
NW = 32
CHUNK = 16384
UNROLL = 8


def _quantize(v):
    t = jnp.minimum(jnp.maximum(v * float(NB - 1), 0.0), float(NB - 1))
    return t.astype(jnp.int32)


def _make_he(nimg, pix):
    ipw = nimg // NW
    nchunk = pix // CHUNK
    scale = 1.0 / float(pix)
    mesh = plsc.VectorSubcoreMesh(core_axis_name="c", subcore_axis_name="s")

    @functools.partial(
        pl.kernel,
        mesh=mesh,
        out_type=jax.ShapeDtypeStruct((nimg * pix,), jnp.float32),
        compiler_params=pltpu.CompilerParams(needs_layout_passes=False),
        scratch_types=[
            pltpu.VMEM((2 * CHUNK,), jnp.float32),
            pltpu.VMEM((2 * CHUNK,), jnp.float32),
            pltpu.VMEM((L * NB,), jnp.float32),
            pltpu.VMEM((NB,), jnp.float32),
            pltpu.SemaphoreType.DMA,
            pltpu.SemaphoreType.DMA,
        ],
    )
    def he(x_hbm, out_hbm, inbuf, outbuf, hist, cdf, insem, outsem):
        wid = lax.axis_index("s") * 2 + lax.axis_index("c")
        lane_base = lax.iota(jnp.int32, L) * NB
        ones = jnp.ones((L,), jnp.float32)
        zeros = jnp.zeros((L,), jnp.float32)

        def start_in(base, ch, slot):
            pltpu.async_copy(x_hbm.at[pl.ds(base + ch * CHUNK, CHUNK)],
                             inbuf.at[pl.ds(slot * CHUNK, CHUNK)], insem)

        def wait_in(base, slot):
            pltpu.make_async_copy(x_hbm.at[pl.ds(base, CHUNK)],
                                  inbuf.at[pl.ds(slot * CHUNK, CHUNK)],
                                  insem).wait()

        for img in range(ipw):
            base = (wid * ipw + img) * pix

            @plsc.parallel_loop(0, L * NB, L, unroll=UNROLL)
            def _(i):
                hist[pl.ds(i, L)] = zeros

            start_in(base, 0, 0)

            def pair1(p, _):
                for sub in range(2):
                    ch = p * 2 + sub
                    wait_in(base, sub)

                    @pl.when(ch + 1 < nchunk)
                    def _():
                        start_in(base, ch + 1, 1 - sub)

                    off = sub * CHUNK

                    @plsc.parallel_loop(0, CHUNK, L, unroll=UNROLL)
                    def _(i):
                        q = _quantize(inbuf[pl.ds(off + i, L)])
                        plsc.addupdate_scatter(hist, [lane_base + q], ones)
                return 0
            lax.fori_loop(0, nchunk // 2, pair1, 0)

            carry = jnp.float32(0.0)
            for g in range(NB // L):
                acc = hist[pl.ds(g * L, L)]
                for r in range(1, L):
                    acc = acc + hist[pl.ds(r * NB + g * L, L)]
                cs = jnp.cumsum(acc) + carry
                carry = jnp.max(cs)
                cdf[pl.ds(g * L, L)] = cs * scale

            start_in(base, 0, 0)

            def pair2(p, _):
                for sub in range(2):
                    ch = p * 2 + sub
                    wait_in(base, sub)

                    @pl.when(ch + 1 < nchunk)
                    def _():
                        start_in(base, ch + 1, 1 - sub)

                    @pl.when(ch >= 2)
                    def _():
                        pltpu.make_async_copy(
                            outbuf.at[pl.ds(sub * CHUNK, CHUNK)],
                            out_hbm.at[pl.ds(base, CHUNK)],
                            outsem).wait()

                    off = sub * CHUNK

                    @plsc.parallel_loop(0, CHUNK, L, unroll=UNROLL)
                    def _(i):
                        q = _quantize(inbuf[pl.ds(off + i, L)])
                        outbuf[pl.ds(off + i, L)] = plsc.load_gather(cdf, [q])

                    pltpu.async_copy(
                        outbuf.at[pl.ds(sub * CHUNK, CHUNK)],
                        out_hbm.at[pl.ds(base + ch * CHUNK, CHUNK)],
                        outsem)
                return 0
            lax.fori_loop(0, nchunk // 2, pair2, 0)

            for sub in range(2):
                pltpu.make_async_copy(
                    outbuf.at[pl.ds(sub * CHUNK, CHUNK)],
                    out_hbm.at[pl.ds(base, CHUNK)],
                    outsem).wait()

    return he


def kernel(x):
    b, c, h, w = x.shape
    nimg, pix = b * c, h * w
    y = _make_he(nimg, pix)(x.reshape(nimg * pix))
    return y.reshape(b, c, h, w)
